# Initial kernel scaffold; baseline (speedup 1.0000x reference)
import jax
import jax.numpy as jnp
from jax.experimental import pallas as pl


def kernel(inputs, edge_index, support_vals, W1, w2, b2, w3, b3, W, B):
    raise NotImplementedError("write your pallas kernel here")


# speedup vs baseline: 23.7804x; 23.7804x over previous
import functools

import jax
import jax.numpy as jnp
from jax import lax
from jax.experimental import pallas as pl
from jax.experimental.pallas import tpu as pltpu
from jax.experimental.pallas import tpu_sc as plsc

N = 10000
E = 330000
D = 128
NW = 32
NB = 81
KB = 128
CHUNK = NB * KB
E_PAD = NW * CHUNK
R_PAD = 10240
SA_PAD = 10112
ROWS_PER_TILE = R_PAD // 16


def _pre_body(x_ref, w1_ref, w2_ref, b2_ref, w3_ref, b3_ref, w_ref,
              val_ref, sa1_ref, sa2_ref):
    x = x_ref[...]
    val_ref[...] = jnp.dot(x, w_ref[...], preferred_element_type=jnp.float32)
    u2 = jnp.dot(w1_ref[...], w2_ref[...], preferred_element_type=jnp.float32)
    u3 = jnp.dot(w1_ref[...], w3_ref[...], preferred_element_type=jnp.float32)
    sa1_ref[...] = jnp.dot(x, u2, preferred_element_type=jnp.float32) + b2_ref[...]
    sa2_ref[...] = jnp.dot(x, u3, preferred_element_type=jnp.float32) + b3_ref[...]


def _pre(x, W1, w2r, b2r, w3r, b3r, W):
    blk = 2000
    return pl.pallas_call(
        _pre_body,
        grid=(N // blk,),
        in_specs=[
            pl.BlockSpec((blk, D), lambda i: (i, 0)),
            pl.BlockSpec((D, D), lambda i: (0, 0)),
            pl.BlockSpec((D, 1), lambda i: (0, 0)),
            pl.BlockSpec((1, 1), lambda i: (0, 0)),
            pl.BlockSpec((D, 1), lambda i: (0, 0)),
            pl.BlockSpec((1, 1), lambda i: (0, 0)),
            pl.BlockSpec((D, D), lambda i: (0, 0)),
        ],
        out_specs=[
            pl.BlockSpec((blk, D), lambda i: (i, 0)),
            pl.BlockSpec((blk, 1), lambda i: (i, 0)),
            pl.BlockSpec((blk, 1), lambda i: (i, 0)),
        ],
        out_shape=[
            jax.ShapeDtypeStruct((N, D), jnp.float32),
            jax.ShapeDtypeStruct((N, 1), jnp.float32),
            jax.ShapeDtypeStruct((N, 1), jnp.float32),
        ],
    )(x, W1, w2r, b2r, w3r, b3r, W)


def _sc_body(rows3, cols3, sup3, sa1_h, sa2_h, val_h, acc_h, den_h,
             sa1_v, sa2_v, idx_r, idx_c, supb, exb, vrows,
             acc_sh, den_sh):
    c = lax.axis_index("c")
    s = lax.axis_index("s")
    wid = s * 2 + c

    pltpu.sync_copy(sa1_h, sa1_v)
    pltpu.sync_copy(sa2_h, sa2_v)

    zero = jnp.zeros((16,), jnp.float32)

    def _zr(i, carry):
        for j in range(8):
            vrows[i, pl.ds(j * 16, 16)] = zero
        return carry

    lax.fori_loop(0, KB, _zr, 0)
    for j in range(8):
        exb[pl.ds(j * 16, 16)] = zero
    for k in range(5):
        pltpu.sync_copy(vrows, acc_sh.at[pl.ds((s * 5 + k) * 128, 128)])
        pltpu.sync_copy(exb, den_sh.at[pl.ds((s * 5 + k) * 128, 128)])
    plsc.subcore_barrier()

    def _batch(b, carry):
        off = (wid * NB + b) * KB
        pltpu.sync_copy(rows3.at[wid, b], idx_r)
        pltpu.sync_copy(cols3.at[wid, b], idx_c)
        pltpu.sync_copy(sup3.at[wid, b], supb)
        for j in range(8):
            r16 = idx_r[pl.ds(j * 16, 16)]
            c16 = idx_c[pl.ds(j * 16, 16)]
            s1 = plsc.load_gather(sa1_v, [r16])
            s2 = plsc.load_gather(sa2_v, [c16])
            sp = supb[pl.ds(j * 16, 16)]
            e = sp * s1 + sp * s2
            e = jnp.maximum(e, e * 0.2)
            ex = jnp.exp(e)
            eid = off + j * 16 + lax.iota(jnp.int32, 16)
            ex = jnp.where(eid < E, ex, 0.0)
            exb[pl.ds(j * 16, 16)] = ex
        pltpu.sync_copy(val_h.at[idx_c], vrows)

        def _scale(g, carry2):
            ex16 = exb[pl.ds(g * 16, 16)]
            base = g * 16
            for l in range(16):
                ex_s = ex16[l]
                for j in range(8):
                    vrows[base + l, pl.ds(j * 16, 16)] = (
                        vrows[base + l, pl.ds(j * 16, 16)] * ex_s)
            return carry2

        lax.fori_loop(0, KB // 16, _scale, 0)
        pltpu.sync_copy(vrows, acc_sh.at[idx_r], add=True)
        pltpu.sync_copy(exb, den_sh.at[idx_r], add=True)
        return carry

    lax.fori_loop(0, NB, _batch, 0)
    plsc.subcore_barrier()

    pltpu.sync_copy(acc_sh.at[pl.ds(s * ROWS_PER_TILE, ROWS_PER_TILE)],
                    acc_h.at[c, pl.ds(s * ROWS_PER_TILE, ROWS_PER_TILE)])
    pltpu.sync_copy(den_sh.at[pl.ds(s * ROWS_PER_TILE, ROWS_PER_TILE)],
                    den_h.at[c, pl.ds(s * ROWS_PER_TILE, ROWS_PER_TILE)])


_sc_call = functools.partial(
    pl.kernel,
    mesh=plsc.VectorSubcoreMesh(core_axis_name="c", subcore_axis_name="s"),
    compiler_params=pltpu.CompilerParams(needs_layout_passes=False),
    out_type=[
        jax.ShapeDtypeStruct((2, R_PAD, D), jnp.float32),
        jax.ShapeDtypeStruct((2, R_PAD), jnp.float32),
    ],
    scratch_types=[
        pltpu.VMEM((SA_PAD,), jnp.float32),
        pltpu.VMEM((SA_PAD,), jnp.float32),
        pltpu.VMEM((KB,), jnp.int32),
        pltpu.VMEM((KB,), jnp.int32),
        pltpu.VMEM((KB,), jnp.float32),
        pltpu.VMEM((KB,), jnp.float32),
        pltpu.VMEM((KB, D), jnp.float32),
        pltpu.VMEM_SHARED((R_PAD, D), jnp.float32),
        pltpu.VMEM_SHARED((R_PAD,), jnp.float32),
    ],
)(_sc_body)


def _post_body(acc_ref, den_ref, b_ref, out_ref):
    a = acc_ref[0] + acc_ref[1]
    d = den_ref[0] + den_ref[1]
    out_ref[...] = a / d + b_ref[...]


def _post(acc, den3, Bp):
    blk = 512
    return pl.pallas_call(
        _post_body,
        grid=(R_PAD // blk,),
        in_specs=[
            pl.BlockSpec((2, blk, D), lambda i: (0, i, 0)),
            pl.BlockSpec((2, blk, 1), lambda i: (0, i, 0)),
            pl.BlockSpec((blk, D), lambda i: (i, 0)),
        ],
        out_specs=pl.BlockSpec((blk, D), lambda i: (i, 0)),
        out_shape=jax.ShapeDtypeStruct((R_PAD, D), jnp.float32),
    )(acc, den3, Bp)


def kernel(inputs, edge_index, support_vals, W1, w2, b2, w3, b3, W, B):
    x = inputs.astype(jnp.float32)
    rows = edge_index[0].astype(jnp.int32)
    cols = edge_index[1].astype(jnp.int32)
    pad = E_PAD - E
    rows3 = jnp.pad(rows, (0, pad)).reshape(NW, NB, KB)
    cols3 = jnp.pad(cols, (0, pad)).reshape(NW, NB, KB)
    sup3 = jnp.pad(support_vals.astype(jnp.float32), (0, pad)).reshape(NW, NB, KB)

    val, sa1, sa2 = _pre(x, W1, w2, b2.reshape(1, 1), w3, b3.reshape(1, 1), W)
    sa_pad = (0, SA_PAD - N)
    acc, den = _sc_call(rows3, cols3, sup3,
                        jnp.pad(sa1.reshape(N), sa_pad),
                        jnp.pad(sa2.reshape(N), sa_pad), val)
    Bp = jnp.pad(B, ((0, R_PAD - N), (0, 0)))
    out = _post(acc, den.reshape(2, R_PAD, 1), Bp)
    return out[:N]


# speedup vs baseline: 34.0584x; 1.4322x over previous
import functools

import jax
import jax.numpy as jnp
from jax import lax
from jax.experimental import pallas as pl
from jax.experimental.pallas import tpu as pltpu
from jax.experimental.pallas import tpu_sc as plsc

N = 10000
E = 330000
D = 128
NW = 32
NB = 108
KB = 96
CHUNK = NB * KB
E_PAD = NW * CHUNK
R_PAD = 10240
SA_PAD = 10112
ROWS_PER_TILE = R_PAD // 16


def _pre_body(x_ref, w1_ref, w2_ref, b2_ref, w3_ref, b3_ref, w_ref,
              val_ref, sa1_ref, sa2_ref):
    x = x_ref[...]
    val_ref[...] = jnp.dot(x, w_ref[...], preferred_element_type=jnp.float32)
    u2 = jnp.dot(w1_ref[...], w2_ref[...], preferred_element_type=jnp.float32)
    u3 = jnp.dot(w1_ref[...], w3_ref[...], preferred_element_type=jnp.float32)
    sa1_ref[...] = jnp.dot(x, u2, preferred_element_type=jnp.float32) + b2_ref[...]
    sa2_ref[...] = jnp.dot(x, u3, preferred_element_type=jnp.float32) + b3_ref[...]


def _pre(x, W1, w2r, b2r, w3r, b3r, W):
    blk = 2000
    return pl.pallas_call(
        _pre_body,
        grid=(N // blk,),
        in_specs=[
            pl.BlockSpec((blk, D), lambda i: (i, 0)),
            pl.BlockSpec((D, D), lambda i: (0, 0)),
            pl.BlockSpec((D, 1), lambda i: (0, 0)),
            pl.BlockSpec((1, 1), lambda i: (0, 0)),
            pl.BlockSpec((D, 1), lambda i: (0, 0)),
            pl.BlockSpec((1, 1), lambda i: (0, 0)),
            pl.BlockSpec((D, D), lambda i: (0, 0)),
        ],
        out_specs=[
            pl.BlockSpec((blk, D), lambda i: (i, 0)),
            pl.BlockSpec((blk, 1), lambda i: (i, 0)),
            pl.BlockSpec((blk, 1), lambda i: (i, 0)),
        ],
        out_shape=[
            jax.ShapeDtypeStruct((N, D), jnp.float32),
            jax.ShapeDtypeStruct((N, 1), jnp.float32),
            jax.ShapeDtypeStruct((N, 1), jnp.float32),
        ],
    )(x, W1, w2r, b2r, w3r, b3r, W)


def _sc_body(edata, sa1_h, sa2_h, val_h, acc_h, den_h,
             sa1_v, sa2_v, pk0, pk1, exb0, exb1, vr0, vr1,
             acc_sh, den_sh, sem_g0, sem_g1, sem_pk):
    c = lax.axis_index("c")
    s = lax.axis_index("s")
    wid = s * 2 + c

    pltpu.sync_copy(sa1_h, sa1_v)
    pltpu.sync_copy(sa2_h, sa2_v)

    zero = jnp.zeros((16,), jnp.float32)

    def _zr(i, carry):
        for j in range(8):
            vr0[i, pl.ds(j * 16, 16)] = zero
        return carry

    lax.fori_loop(0, 64, _zr, 0)
    for j in range(KB // 16):
        exb0[pl.ds(j * 16, 16)] = zero
    for k in range(10):
        pltpu.sync_copy(vr0.at[pl.ds(0, 64)],
                        acc_sh.at[pl.ds(s * ROWS_PER_TILE + k * 64, 64)])
        pltpu.sync_copy(exb0.at[pl.ds(0, 64)],
                        den_sh.at[pl.ds(s * ROWS_PER_TILE + k * 64, 64)])
    plsc.subcore_barrier()

    bufs = ((pk0, exb0, vr0, sem_g0), (pk1, exb1, vr1, sem_g1))

    pltpu.sync_copy(edata.at[wid, 0], pk0)
    pltpu.async_copy(val_h.at[pk0.at[1]], vr0, sem_g0)

    def _pair(i, carry):
        for half in range(2):
            b = 2 * i + half
            pk, exb, vr, sem_g = bufs[half]
            pk_n, _, vr_n, sem_g_n = bufs[1 - half]
            @pl.when(b + 1 < NB)
            def _():
                pltpu.async_copy(edata.at[wid, b + 1], pk_n, sem_pk)

            off = (wid * NB + b) * KB
            for j in range(KB // 16):
                r16 = pk[0, pl.ds(j * 16, 16)]
                c16 = pk[1, pl.ds(j * 16, 16)]
                s1 = plsc.load_gather(sa1_v, [r16])
                s2 = plsc.load_gather(sa2_v, [c16])
                sp = plsc.bitcast(pk[2, pl.ds(j * 16, 16)], jnp.float32)
                e = sp * s1 + sp * s2
                e = jnp.maximum(e, e * 0.2)
                ex = jnp.exp(e)
                eid = off + j * 16 + lax.iota(jnp.int32, 16)
                ex = jnp.where(eid < E, ex, 0.0)
                exb[pl.ds(j * 16, 16)] = ex

            @pl.when(b + 1 < NB)
            def _():
                pltpu.make_async_copy(edata.at[wid, 0], pk_n, sem_pk).wait()
                pltpu.async_copy(val_h.at[pk_n.at[1]], vr_n, sem_g_n)

            pltpu.make_async_copy(val_h.at[pl.ds(0, KB)], vr, sem_g).wait()

            def _scale(g, carry2):
                ex16 = exb[pl.ds(g * 16, 16)]
                base = g * 16
                for l in range(16):
                    ex_s = ex16[l]
                    for j in range(8):
                        vr[base + l, pl.ds(j * 16, 16)] = (
                            vr[base + l, pl.ds(j * 16, 16)] * ex_s)
                return carry2

            lax.fori_loop(0, KB // 16, _scale, 0)
            pltpu.sync_copy(vr, acc_sh.at[pk.at[0]], add=True)
            pltpu.sync_copy(exb, den_sh.at[pk.at[0]], add=True)
        return carry

    lax.fori_loop(0, NB // 2, _pair, 0)
    plsc.subcore_barrier()

    pltpu.sync_copy(acc_sh.at[pl.ds(s * ROWS_PER_TILE, ROWS_PER_TILE)],
                    acc_h.at[c, pl.ds(s * ROWS_PER_TILE, ROWS_PER_TILE)])
    pltpu.sync_copy(den_sh.at[pl.ds(s * ROWS_PER_TILE, ROWS_PER_TILE)],
                    den_h.at[c, pl.ds(s * ROWS_PER_TILE, ROWS_PER_TILE)])


_sc_call = functools.partial(
    pl.kernel,
    mesh=plsc.VectorSubcoreMesh(core_axis_name="c", subcore_axis_name="s"),
    compiler_params=pltpu.CompilerParams(needs_layout_passes=False),
    out_type=[
        jax.ShapeDtypeStruct((2, R_PAD, D), jnp.float32),
        jax.ShapeDtypeStruct((2, R_PAD), jnp.float32),
    ],
    scratch_types=[
        pltpu.VMEM((SA_PAD,), jnp.float32),
        pltpu.VMEM((SA_PAD,), jnp.float32),
        pltpu.VMEM((3, KB), jnp.int32),
        pltpu.VMEM((3, KB), jnp.int32),
        pltpu.VMEM((KB,), jnp.float32),
        pltpu.VMEM((KB,), jnp.float32),
        pltpu.VMEM((KB, D), jnp.float32),
        pltpu.VMEM((KB, D), jnp.float32),
        pltpu.VMEM_SHARED((R_PAD, D), jnp.float32),
        pltpu.VMEM_SHARED((R_PAD,), jnp.float32),
        pltpu.SemaphoreType.DMA,
        pltpu.SemaphoreType.DMA,
        pltpu.SemaphoreType.DMA,
    ],
)(_sc_body)


def _post_body(acc_ref, den_ref, b_ref, out_ref):
    a = acc_ref[0] + acc_ref[1]
    d = den_ref[0] + den_ref[1]
    out_ref[...] = a / d + b_ref[...]


def _post(acc, den3, Bp):
    blk = 512
    return pl.pallas_call(
        _post_body,
        grid=(R_PAD // blk,),
        in_specs=[
            pl.BlockSpec((2, blk, D), lambda i: (0, i, 0)),
            pl.BlockSpec((2, blk, 1), lambda i: (0, i, 0)),
            pl.BlockSpec((blk, D), lambda i: (i, 0)),
        ],
        out_specs=pl.BlockSpec((blk, D), lambda i: (i, 0)),
        out_shape=jax.ShapeDtypeStruct((R_PAD, D), jnp.float32),
    )(acc, den3, Bp)


def kernel(inputs, edge_index, support_vals, W1, w2, b2, w3, b3, W, B):
    x = inputs.astype(jnp.float32)
    rows = edge_index[0].astype(jnp.int32)
    cols = edge_index[1].astype(jnp.int32)
    pad = E_PAD - E
    rows3 = jnp.pad(rows, (0, pad)).reshape(NW, NB, KB)
    cols3 = jnp.pad(cols, (0, pad)).reshape(NW, NB, KB)
    supbits3 = lax.bitcast_convert_type(
        jnp.pad(support_vals.astype(jnp.float32), (0, pad)),
        jnp.int32).reshape(NW, NB, KB)
    edata = jnp.stack([rows3, cols3, supbits3], axis=2)

    val, sa1, sa2 = _pre(x, W1, w2, b2.reshape(1, 1), w3, b3.reshape(1, 1), W)
    sa_pad = (0, SA_PAD - N)
    acc, den = _sc_call(edata,
                        jnp.pad(sa1.reshape(N), sa_pad),
                        jnp.pad(sa2.reshape(N), sa_pad), val)
    Bp = jnp.pad(B, ((0, R_PAD - N), (0, 0)))
    out = _post(acc, den.reshape(2, R_PAD, 1), Bp)
    return out[:N]


# speedup vs baseline: 37.3105x; 1.0955x over previous
import functools

import jax
import jax.numpy as jnp
from jax import lax
from jax.experimental import pallas as pl
from jax.experimental.pallas import tpu as pltpu
from jax.experimental.pallas import tpu_sc as plsc

N = 10000
E = 330000
D = 128
NW = 32
NB = 108
KB = 96
CHUNK = NB * KB
E_PAD = NW * CHUNK
R_PAD = 10240
SA_PAD = 10112
ROWS_PER_TILE = R_PAD // 16


def _pre_body(x_ref, w1_ref, w2_ref, b2_ref, w3_ref, b3_ref, w_ref,
              val_ref, sa1_ref, sa2_ref):
    x = x_ref[...]
    val_ref[...] = jnp.dot(x, w_ref[...], preferred_element_type=jnp.float32)
    u2 = jnp.dot(w1_ref[...], w2_ref[...], preferred_element_type=jnp.float32)
    u3 = jnp.dot(w1_ref[...], w3_ref[...], preferred_element_type=jnp.float32)
    sa1_ref[...] = jnp.dot(x, u2, preferred_element_type=jnp.float32) + b2_ref[...]
    sa2_ref[...] = jnp.dot(x, u3, preferred_element_type=jnp.float32) + b3_ref[...]


def _pre(x, W1, w2r, b2r, w3r, b3r, W):
    blk = 2000
    return pl.pallas_call(
        _pre_body,
        grid=(N // blk,),
        in_specs=[
            pl.BlockSpec((blk, D), lambda i: (i, 0)),
            pl.BlockSpec((D, D), lambda i: (0, 0)),
            pl.BlockSpec((D, 1), lambda i: (0, 0)),
            pl.BlockSpec((1, 1), lambda i: (0, 0)),
            pl.BlockSpec((D, 1), lambda i: (0, 0)),
            pl.BlockSpec((1, 1), lambda i: (0, 0)),
            pl.BlockSpec((D, D), lambda i: (0, 0)),
        ],
        out_specs=[
            pl.BlockSpec((blk, D), lambda i: (i, 0)),
            pl.BlockSpec((blk, 1), lambda i: (i, 0)),
            pl.BlockSpec((blk, 1), lambda i: (i, 0)),
        ],
        out_shape=[
            jax.ShapeDtypeStruct((N, D), jnp.float32),
            jax.ShapeDtypeStruct((N, 1), jnp.float32),
            jax.ShapeDtypeStruct((N, 1), jnp.float32),
        ],
    )(x, W1, w2r, b2r, w3r, b3r, W)


def _sc_body(edata, sa1_h, sa2_h, val_h, acc_h, den_h,
             sa1_v, sa2_v, pk0, pk1, idx_s0, idx_s1, exb0, exb1, vr0, vr1,
             acc_sh, den_sh, sem_g0, sem_g1, sem_s0, sem_s1, sem_pk):
    c = lax.axis_index("c")
    s = lax.axis_index("s")
    wid = s * 2 + c

    pltpu.sync_copy(sa1_h, sa1_v)
    pltpu.sync_copy(sa2_h, sa2_v)

    zero = jnp.zeros((16,), jnp.float32)

    def _zr(i, carry):
        for j in range(8):
            vr0[i, pl.ds(j * 16, 16)] = zero
        return carry

    lax.fori_loop(0, 64, _zr, 0)
    for j in range(KB // 16):
        exb0[pl.ds(j * 16, 16)] = zero
    for k in range(10):
        pltpu.sync_copy(vr0.at[pl.ds(0, 64)],
                        acc_sh.at[pl.ds(s * ROWS_PER_TILE + k * 64, 64)])
        pltpu.sync_copy(exb0.at[pl.ds(0, 64)],
                        den_sh.at[pl.ds(s * ROWS_PER_TILE + k * 64, 64)])
    plsc.subcore_barrier()

    bufs = ((pk0, idx_s0, exb0, vr0, sem_g0, sem_s0),
            (pk1, idx_s1, exb1, vr1, sem_g1, sem_s1))

    pltpu.sync_copy(edata.at[wid, 0], pk0)
    pltpu.async_copy(val_h.at[pk0.at[1]], vr0, sem_g0)

    def _pair(i, carry):
        for half in range(2):
            b = 2 * i + half
            pk, idx_s, exb, vr, sem_g, sem_s = bufs[half]
            pk_n, _, exb_n, vr_n, sem_g_n, sem_s_n = bufs[1 - half]
            @pl.when(b + 1 < NB)
            def _():
                pltpu.async_copy(edata.at[wid, b + 1], pk_n, sem_pk)

            off = (wid * NB + b) * KB
            for j in range(KB // 16):
                r16 = pk[0, pl.ds(j * 16, 16)]
                idx_s[pl.ds(j * 16, 16)] = r16
                c16 = pk[1, pl.ds(j * 16, 16)]
                s1 = plsc.load_gather(sa1_v, [r16])
                s2 = plsc.load_gather(sa2_v, [c16])
                sp = plsc.bitcast(pk[2, pl.ds(j * 16, 16)], jnp.float32)
                e = sp * s1 + sp * s2
                e = jnp.maximum(e, e * 0.2)
                ex = jnp.exp(e)
                eid = off + j * 16 + lax.iota(jnp.int32, 16)
                ex = jnp.where(eid < E, ex, 0.0)
                exb[pl.ds(j * 16, 16)] = ex

            @pl.when(b + 1 < NB)
            def _():
                @pl.when(b >= 1)
                def _():
                    pltpu.make_async_copy(
                        val_h.at[pl.ds(0, KB)], vr_n, sem_s_n).wait()
                    pltpu.make_async_copy(
                        sa1_h.at[pl.ds(0, KB)], exb_n, sem_s_n).wait()
                pltpu.make_async_copy(edata.at[wid, 0], pk_n, sem_pk).wait()
                pltpu.async_copy(val_h.at[pk_n.at[1]], vr_n, sem_g_n)

            pltpu.make_async_copy(val_h.at[pl.ds(0, KB)], vr, sem_g).wait()

            def _scale(g, carry2):
                ex16 = exb[pl.ds(g * 16, 16)]
                base = g * 16
                for l in range(16):
                    ex_s = ex16[l]
                    for j in range(8):
                        vr[base + l, pl.ds(j * 16, 16)] = (
                            vr[base + l, pl.ds(j * 16, 16)] * ex_s)
                return carry2

            lax.fori_loop(0, KB // 16, _scale, 0)
            pltpu.async_copy(vr, acc_sh.at[idx_s], sem_s, add=True)
            pltpu.async_copy(exb, den_sh.at[idx_s], sem_s, add=True)
        return carry

    lax.fori_loop(0, NB // 2, _pair, 0)
    for _, _, exb, vr, _, sem_s in bufs:
        pltpu.make_async_copy(val_h.at[pl.ds(0, KB)], vr, sem_s).wait()
        pltpu.make_async_copy(sa1_h.at[pl.ds(0, KB)], exb, sem_s).wait()
    plsc.subcore_barrier()

    pltpu.sync_copy(acc_sh.at[pl.ds(s * ROWS_PER_TILE, ROWS_PER_TILE)],
                    acc_h.at[c, pl.ds(s * ROWS_PER_TILE, ROWS_PER_TILE)])
    pltpu.sync_copy(den_sh.at[pl.ds(s * ROWS_PER_TILE, ROWS_PER_TILE)],
                    den_h.at[c, pl.ds(s * ROWS_PER_TILE, ROWS_PER_TILE)])


_sc_call = functools.partial(
    pl.kernel,
    mesh=plsc.VectorSubcoreMesh(core_axis_name="c", subcore_axis_name="s"),
    compiler_params=pltpu.CompilerParams(needs_layout_passes=False),
    out_type=[
        jax.ShapeDtypeStruct((2, R_PAD, D), jnp.float32),
        jax.ShapeDtypeStruct((2, R_PAD), jnp.float32),
    ],
    scratch_types=[
        pltpu.VMEM((SA_PAD,), jnp.float32),
        pltpu.VMEM((SA_PAD,), jnp.float32),
        pltpu.VMEM((3, KB), jnp.int32),
        pltpu.VMEM((3, KB), jnp.int32),
        pltpu.VMEM((KB,), jnp.int32),
        pltpu.VMEM((KB,), jnp.int32),
        pltpu.VMEM((KB,), jnp.float32),
        pltpu.VMEM((KB,), jnp.float32),
        pltpu.VMEM((KB, D), jnp.float32),
        pltpu.VMEM((KB, D), jnp.float32),
        pltpu.VMEM_SHARED((R_PAD, D), jnp.float32),
        pltpu.VMEM_SHARED((R_PAD,), jnp.float32),
        pltpu.SemaphoreType.DMA,
        pltpu.SemaphoreType.DMA,
        pltpu.SemaphoreType.DMA,
        pltpu.SemaphoreType.DMA,
        pltpu.SemaphoreType.DMA,
    ],
)(_sc_body)


def _post_body(acc_ref, den_ref, b_ref, out_ref):
    a = acc_ref[0] + acc_ref[1]
    d = den_ref[0] + den_ref[1]
    out_ref[...] = a / d + b_ref[...]


def _post(acc, den3, Bp):
    blk = 512
    return pl.pallas_call(
        _post_body,
        grid=(R_PAD // blk,),
        in_specs=[
            pl.BlockSpec((2, blk, D), lambda i: (0, i, 0)),
            pl.BlockSpec((2, blk, 1), lambda i: (0, i, 0)),
            pl.BlockSpec((blk, D), lambda i: (i, 0)),
        ],
        out_specs=pl.BlockSpec((blk, D), lambda i: (i, 0)),
        out_shape=jax.ShapeDtypeStruct((R_PAD, D), jnp.float32),
    )(acc, den3, Bp)


def kernel(inputs, edge_index, support_vals, W1, w2, b2, w3, b3, W, B):
    x = inputs.astype(jnp.float32)
    rows = edge_index[0].astype(jnp.int32)
    cols = edge_index[1].astype(jnp.int32)
    pad = E_PAD - E
    rows3 = jnp.pad(rows, (0, pad)).reshape(NW, NB, KB)
    cols3 = jnp.pad(cols, (0, pad)).reshape(NW, NB, KB)
    supbits3 = lax.bitcast_convert_type(
        jnp.pad(support_vals.astype(jnp.float32), (0, pad)),
        jnp.int32).reshape(NW, NB, KB)
    edata = jnp.stack([rows3, cols3, supbits3], axis=2)

    val, sa1, sa2 = _pre(x, W1, w2, b2.reshape(1, 1), w3, b3.reshape(1, 1), W)
    sa_pad = (0, SA_PAD - N)
    acc, den = _sc_call(edata,
                        jnp.pad(sa1.reshape(N), sa_pad),
                        jnp.pad(sa2.reshape(N), sa_pad), val)
    Bp = jnp.pad(B, ((0, R_PAD - N), (0, 0)))
    out = _post(acc, den.reshape(2, R_PAD, 1), Bp)
    return out[:N]


# speedup vs baseline: 40.1990x; 1.0774x over previous
import functools

import jax
import jax.numpy as jnp
from jax import lax
from jax.experimental import pallas as pl
from jax.experimental.pallas import tpu as pltpu
from jax.experimental.pallas import tpu_sc as plsc

N = 10000
E = 330000
D = 128
NW = 32
NB = 108
KB = 96
CHUNK = NB * KB
E_PAD = NW * CHUNK
R_PAD = 10240
SA_PAD = 10112
ROWS_PER_TILE = R_PAD // 16


def _pre_body(x_ref, w1_ref, w2_ref, b2_ref, w3_ref, b3_ref, w_ref,
              val_ref, sa1_ref, sa2_ref):
    x = x_ref[...]
    val_ref[...] = jnp.dot(x, w_ref[...], preferred_element_type=jnp.float32)
    u2 = jnp.dot(w1_ref[...], w2_ref[...], preferred_element_type=jnp.float32)
    u3 = jnp.dot(w1_ref[...], w3_ref[...], preferred_element_type=jnp.float32)
    sa1_ref[...] = jnp.dot(x, u2, preferred_element_type=jnp.float32) + b2_ref[...]
    sa2_ref[...] = jnp.dot(x, u3, preferred_element_type=jnp.float32) + b3_ref[...]


def _pre(x, W1, w2r, b2r, w3r, b3r, W):
    blk = 2000
    return pl.pallas_call(
        _pre_body,
        grid=(N // blk,),
        in_specs=[
            pl.BlockSpec((blk, D), lambda i: (i, 0)),
            pl.BlockSpec((D, D), lambda i: (0, 0)),
            pl.BlockSpec((D, 1), lambda i: (0, 0)),
            pl.BlockSpec((1, 1), lambda i: (0, 0)),
            pl.BlockSpec((D, 1), lambda i: (0, 0)),
            pl.BlockSpec((1, 1), lambda i: (0, 0)),
            pl.BlockSpec((D, D), lambda i: (0, 0)),
        ],
        out_specs=[
            pl.BlockSpec((blk, D), lambda i: (i, 0)),
            pl.BlockSpec((blk, 1), lambda i: (i, 0)),
            pl.BlockSpec((blk, 1), lambda i: (i, 0)),
        ],
        out_shape=[
            jax.ShapeDtypeStruct((N, D), jnp.float32),
            jax.ShapeDtypeStruct((N, 1), jnp.float32),
            jax.ShapeDtypeStruct((N, 1), jnp.float32),
        ],
    )(x, W1, w2r, b2r, w3r, b3r, W)


def _sc_body(rows2, cols2, sup2, sa1_h, sa2_h, val_h, acc_h, den_h,
             sa1_v, sa2_v, pk0, pk1, idx_s0, idx_s1, exb0, exb1, vr0, vr1,
             acc_sh, den_sh, sem_g0, sem_g1, sem_s0, sem_s1, sem_pk):
    c = lax.axis_index("c")
    s = lax.axis_index("s")
    wid = s * 2 + c

    pltpu.sync_copy(sa1_h, sa1_v)
    pltpu.sync_copy(sa2_h, sa2_v)

    zero = jnp.zeros((16,), jnp.float32)

    def _zr(i, carry):
        for j in range(8):
            vr0[i, pl.ds(j * 16, 16)] = zero
        return carry

    lax.fori_loop(0, 64, _zr, 0)
    for j in range(KB // 16):
        exb0[pl.ds(j * 16, 16)] = zero
    for k in range(10):
        pltpu.sync_copy(vr0.at[pl.ds(0, 64)],
                        acc_sh.at[pl.ds(s * ROWS_PER_TILE + k * 64, 64)])
        pltpu.sync_copy(exb0.at[pl.ds(0, 64)],
                        den_sh.at[pl.ds(s * ROWS_PER_TILE + k * 64, 64)])
    plsc.subcore_barrier()

    bufs = ((pk0, idx_s0, exb0, vr0, sem_g0, sem_s0),
            (pk1, idx_s1, exb1, vr1, sem_g1, sem_s1))

    pltpu.sync_copy(rows2.at[wid, 0], pk0.at[0])
    pltpu.sync_copy(cols2.at[wid, 0], pk0.at[1])
    pltpu.sync_copy(sup2.at[wid, 0], pk0.at[2])
    pltpu.async_copy(val_h.at[pk0.at[1]], vr0, sem_g0)

    def _pair(i, carry):
        for half in range(2):
            b = 2 * i + half
            pk, idx_s, exb, vr, sem_g, sem_s = bufs[half]
            pk_n, _, exb_n, vr_n, sem_g_n, sem_s_n = bufs[1 - half]
            @pl.when(b + 1 < NB)
            def _():
                pltpu.async_copy(rows2.at[wid, b + 1], pk_n.at[0], sem_pk)
                pltpu.async_copy(cols2.at[wid, b + 1], pk_n.at[1], sem_pk)
                pltpu.async_copy(sup2.at[wid, b + 1], pk_n.at[2], sem_pk)

            off = (wid * NB + b) * KB
            for j in range(KB // 16):
                r16 = pk[0, pl.ds(j * 16, 16)]
                idx_s[pl.ds(j * 16, 16)] = r16
                c16 = pk[1, pl.ds(j * 16, 16)]
                s1 = plsc.load_gather(sa1_v, [r16])
                s2 = plsc.load_gather(sa2_v, [c16])
                sp = plsc.bitcast(pk[2, pl.ds(j * 16, 16)], jnp.float32)
                e = sp * s1 + sp * s2
                e = jnp.maximum(e, e * 0.2)
                ex = jnp.exp(e)
                eid = off + j * 16 + lax.iota(jnp.int32, 16)
                ex = jnp.where(eid < E, ex, 0.0)
                exb[pl.ds(j * 16, 16)] = ex

            @pl.when(b + 1 < NB)
            def _():
                @pl.when(b >= 1)
                def _():
                    pltpu.make_async_copy(
                        val_h.at[pl.ds(0, KB)], vr_n, sem_s_n).wait()
                    pltpu.make_async_copy(
                        sa1_h.at[pl.ds(0, KB)], exb_n, sem_s_n).wait()
                for _k in range(3):
                    pltpu.make_async_copy(
                        rows2.at[wid, 0], pk_n.at[_k], sem_pk).wait()
                pltpu.async_copy(val_h.at[pk_n.at[1]], vr_n, sem_g_n)

            pltpu.make_async_copy(val_h.at[pl.ds(0, KB)], vr, sem_g).wait()

            def _scale(g, carry2):
                ex16 = exb[pl.ds(g * 16, 16)]
                base = g * 16
                for l in range(16):
                    ex_s = ex16[l]
                    for j in range(8):
                        vr[base + l, pl.ds(j * 16, 16)] = (
                            vr[base + l, pl.ds(j * 16, 16)] * ex_s)
                return carry2

            lax.fori_loop(0, KB // 16, _scale, 0)
            pltpu.async_copy(vr, acc_sh.at[idx_s], sem_s, add=True)
            pltpu.async_copy(exb, den_sh.at[idx_s], sem_s, add=True)
        return carry

    lax.fori_loop(0, NB // 2, _pair, 0)
    for _, _, exb, vr, _, sem_s in bufs:
        pltpu.make_async_copy(val_h.at[pl.ds(0, KB)], vr, sem_s).wait()
        pltpu.make_async_copy(sa1_h.at[pl.ds(0, KB)], exb, sem_s).wait()
    plsc.subcore_barrier()

    pltpu.sync_copy(acc_sh.at[pl.ds(s * ROWS_PER_TILE, ROWS_PER_TILE)],
                    acc_h.at[c, pl.ds(s * ROWS_PER_TILE, ROWS_PER_TILE)])
    pltpu.sync_copy(den_sh.at[pl.ds(s * ROWS_PER_TILE, ROWS_PER_TILE)],
                    den_h.at[c, pl.ds(s * ROWS_PER_TILE, ROWS_PER_TILE)])


_sc_call = functools.partial(
    pl.kernel,
    mesh=plsc.VectorSubcoreMesh(core_axis_name="c", subcore_axis_name="s"),
    compiler_params=pltpu.CompilerParams(needs_layout_passes=False),
    out_type=[
        jax.ShapeDtypeStruct((2, R_PAD, D), jnp.float32),
        jax.ShapeDtypeStruct((2, R_PAD), jnp.float32),
    ],
    scratch_types=[
        pltpu.VMEM((SA_PAD,), jnp.float32),
        pltpu.VMEM((SA_PAD,), jnp.float32),
        pltpu.VMEM((3, KB), jnp.int32),
        pltpu.VMEM((3, KB), jnp.int32),
        pltpu.VMEM((KB,), jnp.int32),
        pltpu.VMEM((KB,), jnp.int32),
        pltpu.VMEM((KB,), jnp.float32),
        pltpu.VMEM((KB,), jnp.float32),
        pltpu.VMEM((KB, D), jnp.float32),
        pltpu.VMEM((KB, D), jnp.float32),
        pltpu.VMEM_SHARED((R_PAD, D), jnp.float32),
        pltpu.VMEM_SHARED((R_PAD,), jnp.float32),
        pltpu.SemaphoreType.DMA,
        pltpu.SemaphoreType.DMA,
        pltpu.SemaphoreType.DMA,
        pltpu.SemaphoreType.DMA,
        pltpu.SemaphoreType.DMA,
    ],
)(_sc_body)


def _post_body(acc_ref, den_ref, b_ref, out_ref):
    a = acc_ref[0] + acc_ref[1]
    d = den_ref[0] + den_ref[1]
    out_ref[...] = a / d + b_ref[...]


def _post(acc, den3, B):
    blk = 400
    return pl.pallas_call(
        _post_body,
        grid=(N // blk,),
        in_specs=[
            pl.BlockSpec((2, blk, D), lambda i: (0, i, 0)),
            pl.BlockSpec((2, blk, 1), lambda i: (0, i, 0)),
            pl.BlockSpec((blk, D), lambda i: (i, 0)),
        ],
        out_specs=pl.BlockSpec((blk, D), lambda i: (i, 0)),
        out_shape=jax.ShapeDtypeStruct((N, D), jnp.float32),
    )(acc, den3, B)


def kernel(inputs, edge_index, support_vals, W1, w2, b2, w3, b3, W, B):
    x = inputs.astype(jnp.float32)
    rows = edge_index[0].astype(jnp.int32)
    cols = edge_index[1].astype(jnp.int32)
    pad = E_PAD - E
    rows3 = jnp.pad(rows, (0, pad)).reshape(NW, NB, KB)
    cols3 = jnp.pad(cols, (0, pad)).reshape(NW, NB, KB)
    supbits3 = lax.bitcast_convert_type(
        jnp.pad(support_vals.astype(jnp.float32), (0, pad)),
        jnp.int32).reshape(NW, NB, KB)

    val, sa1, sa2 = _pre(x, W1, w2, b2.reshape(1, 1), w3, b3.reshape(1, 1), W)
    sa_pad = (0, SA_PAD - N)
    acc, den = _sc_call(rows3, cols3, supbits3,
                        jnp.pad(sa1.reshape(N), sa_pad),
                        jnp.pad(sa2.reshape(N), sa_pad), val)
    return _post(acc, den.reshape(2, R_PAD, 1), B)
